# baseline (speedup 1.0000x reference)
import functools

import jax
import jax.numpy as jnp
from jax import lax
from jax.experimental import pallas as pl
from jax.experimental.pallas import tpu as pltpu
from jax.experimental.pallas import tpu_sc as plsc

N = 10000
E = 320000
D = 128

NC = 2
NS = 16
NW = NC * NS

CHUNK = 128
CPW = 80
EP = NW * CPW * CHUNK
NZ = 10240
RPT = NZ // NS

_MESH = plsc.VectorSubcoreMesh(core_axis_name="c", subcore_axis_name="s")


@functools.partial(
    pl.kernel,
    mesh=_MESH,
    out_type=jax.ShapeDtypeStruct((NC, NZ, 16), jnp.float32),
    scratch_types=[
        pltpu.VMEM((CPW, CHUNK), jnp.int32),
        pltpu.VMEM((CHUNK, 16), jnp.float32),
        pltpu.VMEM((CHUNK, 16), jnp.float32),
        pltpu.VMEM_SHARED((NZ, 16), jnp.float32),
    ],
)
def _deg_kernel(dst_hbm, out_hbm, dst_v, ones_v, zeros_v, deg_sh):
    c = lax.axis_index("c")
    s = lax.axis_index("s")
    wid = s * NC + c

    def fill(i, carry):
        ones_v[i, :] = jnp.full((16,), 1.0, jnp.float32)
        zeros_v[i, :] = jnp.zeros((16,), jnp.float32)
        return carry

    lax.fori_loop(0, CHUNK, fill, None)

    def zero_sh(i, carry):
        pltpu.sync_copy(zeros_v, deg_sh.at[pl.ds(s * RPT + i * CHUNK, CHUNK)])
        return carry

    lax.fori_loop(0, RPT // CHUNK, zero_sh, None)
    plsc.subcore_barrier()

    pltpu.sync_copy(dst_hbm.at[pl.ds(wid * CPW, CPW)], dst_v)

    def body(j, carry):
        pltpu.sync_copy(ones_v, deg_sh.at[dst_v.at[j]], add=True)
        return carry

    lax.fori_loop(0, CPW, body, None)
    plsc.subcore_barrier()
    pltpu.sync_copy(deg_sh.at[pl.ds(s * RPT, RPT)],
                    out_hbm.at[c, pl.ds(s * RPT, RPT)])


SUP = 8
NCH = EP // CHUNK
CPT = NCH // NS


@functools.partial(
    pl.kernel,
    mesh=_MESH,
    out_type=jax.ShapeDtypeStruct((NZ, D), jnp.float32),
    scratch_types=[
        pltpu.VMEM((SUP, CHUNK), jnp.int32),
        pltpu.VMEM((SUP, CHUNK), jnp.int32),
        pltpu.VMEM((CHUNK, D), jnp.float32),
        pltpu.VMEM((CHUNK, D), jnp.float32),
        pltpu.VMEM_SHARED((NZ, D), jnp.float32),
        pltpu.SemaphoreType.DMA,
        pltpu.SemaphoreType.DMA,
    ],
)
def _edge_kernel(src_hbm, dst_hbm, y_hbm, out_hbm,
                 src_v, dst_v, rows_a, rows_b, z_sh, sem_a, sem_b):
    c = lax.axis_index("c")
    s = lax.axis_index("s")

    def gather(j, buf, sem):
        return pltpu.make_async_copy(y_hbm.at[src_v.at[j]], buf, sem)

    def group(g, carry):
        base = s * CPT + g * SUP
        pltpu.sync_copy(src_hbm.at[pl.ds(base, SUP)], src_v)
        pltpu.sync_copy(dst_hbm.at[pl.ds(base, SUP)], dst_v)
        gather(0, rows_a, sem_a).start()
        gather(1, rows_b, sem_b).start()

        def body(i, carry2):
            j0 = 2 * i
            j1 = 2 * i + 1
            gather(j0, rows_a, sem_a).wait()
            pltpu.sync_copy(rows_a, z_sh.at[dst_v.at[j0]], add=True)

            @pl.when(j0 + 2 < SUP)
            def _():
                gather(j0 + 2, rows_a, sem_a).start()

            gather(j1, rows_b, sem_b).wait()
            pltpu.sync_copy(rows_b, z_sh.at[dst_v.at[j1]], add=True)

            @pl.when(j1 + 2 < SUP)
            def _():
                gather(j1 + 2, rows_b, sem_b).start()

            return carry2

        lax.fori_loop(0, SUP // 2, body, None)
        return carry

    @pl.when(c == 0)
    def _():
        def zrow(i, carry):
            for l in range(D // 16):
                rows_a[i, pl.ds(l * 16, 16)] = jnp.zeros((16,), jnp.float32)
            return carry

        lax.fori_loop(0, CHUNK, zrow, None)

        def zero_sh(i, carry):
            pltpu.sync_copy(rows_a,
                            z_sh.at[pl.ds(s * RPT + i * CHUNK, CHUNK)])
            return carry

        lax.fori_loop(0, RPT // CHUNK, zero_sh, None)
        plsc.subcore_barrier()
        lax.fori_loop(0, CPT // SUP, group, None)
        plsc.subcore_barrier()
        pltpu.sync_copy(z_sh.at[pl.ds(s * RPT, RPT)],
                        out_hbm.at[pl.ds(s * RPT, RPT)])


BM = 320
GB = NZ // BM
BMO = 400
GBO = N // BMO


def _mm1_body(degp_ref, x_ref, w_ref, y_ref, dinv_ref):
    degp = degp_ref[...]
    deg = degp[0, :, 0] + degp[1, :, 0] + 1.0
    dinv = lax.rsqrt(deg)
    xw = jnp.dot(x_ref[...], w_ref[...], preferred_element_type=jnp.float32)
    y_ref[...] = xw * dinv[:, None]
    dinv_ref[...] = dinv[:, None]


def _mm2_body(zp_ref, y1_ref, dinv_ref, b1_ref, w2_ref, y2_ref):
    dinv = dinv_ref[...]
    t = (zp_ref[...] + y1_ref[...]) * dinv + b1_ref[...]
    h = jnp.maximum(t, 0.0)
    y2_ref[...] = jnp.dot(h, w2_ref[...],
                          preferred_element_type=jnp.float32) * dinv


def _out_body(zp_ref, y2_ref, dinv_ref, b2_ref, o_ref):
    o_ref[...] = (zp_ref[...] + y2_ref[...]) * dinv_ref[...] + b2_ref[...]


_mm1 = pl.pallas_call(
    _mm1_body,
    grid=(GB,),
    in_specs=[
        pl.BlockSpec((NC, BM, 16), lambda i: (0, i, 0)),
        pl.BlockSpec((BM, D), lambda i: (i, 0)),
        pl.BlockSpec((D, D), lambda i: (0, 0)),
    ],
    out_specs=[
        pl.BlockSpec((BM, D), lambda i: (i, 0)),
        pl.BlockSpec((BM, 1), lambda i: (i, 0)),
    ],
    out_shape=[
        jax.ShapeDtypeStruct((NZ, D), jnp.float32),
        jax.ShapeDtypeStruct((NZ, 1), jnp.float32),
    ],
)

_mm2 = pl.pallas_call(
    _mm2_body,
    grid=(GB,),
    in_specs=[
        pl.BlockSpec((BM, D), lambda i: (i, 0)),
        pl.BlockSpec((BM, D), lambda i: (i, 0)),
        pl.BlockSpec((BM, 1), lambda i: (i, 0)),
        pl.BlockSpec((1, D), lambda i: (0, 0)),
        pl.BlockSpec((D, D), lambda i: (0, 0)),
    ],
    out_specs=pl.BlockSpec((BM, D), lambda i: (i, 0)),
    out_shape=jax.ShapeDtypeStruct((NZ, D), jnp.float32),
)

_out = pl.pallas_call(
    _out_body,
    grid=(GBO,),
    in_specs=[
        pl.BlockSpec((BMO, D), lambda i: (i, 0)),
        pl.BlockSpec((BMO, D), lambda i: (i, 0)),
        pl.BlockSpec((BMO, 1), lambda i: (i, 0)),
        pl.BlockSpec((1, D), lambda i: (0, 0)),
    ],
    out_specs=pl.BlockSpec((BMO, D), lambda i: (i, 0)),
    out_shape=jax.ShapeDtypeStruct((N, D), jnp.float32),
)


def kernel(x, edge_index, W1, b1, W2, b2):
    src = edge_index[0]
    dst = edge_index[1]
    pad = EP - E
    x_p = jnp.concatenate([x, jnp.zeros((NZ - N, D), jnp.float32)])
    src_p = jnp.concatenate(
        [src, jnp.zeros((pad,), jnp.int32)]).reshape(EP // CHUNK, CHUNK)
    dump = N + jnp.arange(pad, dtype=jnp.int32) % (NZ - N)
    dst_p = jnp.concatenate([dst, dump]).reshape(EP // CHUNK, CHUNK)

    degp = _deg_kernel(dst_p)
    y1, dinv = _mm1(degp, x_p, W1)
    z1 = _edge_kernel(src_p, dst_p, y1)
    y2 = _mm2(z1, y1, dinv, b1.reshape(1, D), W2)
    z2 = _edge_kernel(src_p, dst_p, y2)
    return _out(z2, y2, dinv, b2.reshape(1, D))


# speedup vs baseline: 1.9960x; 1.9960x over previous
import functools

import jax
import jax.numpy as jnp
from jax import lax
from jax.experimental import pallas as pl
from jax.experimental.pallas import tpu as pltpu
from jax.experimental.pallas import tpu_sc as plsc

N = 10000
E = 320000
D = 128

NC = 2
NS = 16
NW = NC * NS

CHUNK = 128
CPW = 80
EP = NW * CPW * CHUNK
NZ = 10240
RPT = NZ // NS

_MESH = plsc.VectorSubcoreMesh(core_axis_name="c", subcore_axis_name="s")


@functools.partial(
    pl.kernel,
    mesh=_MESH,
    out_type=jax.ShapeDtypeStruct((NC, NZ, 16), jnp.float32),
    scratch_types=[
        pltpu.VMEM((CPW, CHUNK), jnp.int32),
        pltpu.VMEM((CHUNK, 16), jnp.float32),
        pltpu.VMEM((CHUNK, 16), jnp.float32),
        pltpu.VMEM_SHARED((NZ, 16), jnp.float32),
    ],
)
def _deg_kernel(dst_hbm, out_hbm, dst_v, ones_v, zeros_v, deg_sh):
    c = lax.axis_index("c")
    s = lax.axis_index("s")
    wid = s * NC + c

    def fill(i, carry):
        ones_v[i, :] = jnp.full((16,), 1.0, jnp.float32)
        zeros_v[i, :] = jnp.zeros((16,), jnp.float32)
        return carry

    lax.fori_loop(0, CHUNK, fill, None)

    def zero_sh(i, carry):
        pltpu.sync_copy(zeros_v, deg_sh.at[pl.ds(s * RPT + i * CHUNK, CHUNK)])
        return carry

    lax.fori_loop(0, RPT // CHUNK, zero_sh, None)
    plsc.subcore_barrier()

    pltpu.sync_copy(dst_hbm.at[pl.ds(wid * CPW, CPW)], dst_v)

    def body(j, carry):
        pltpu.sync_copy(ones_v, deg_sh.at[dst_v.at[j]], add=True)
        return carry

    lax.fori_loop(0, CPW, body, None)
    plsc.subcore_barrier()
    pltpu.sync_copy(deg_sh.at[pl.ds(s * RPT, RPT)],
                    out_hbm.at[c, pl.ds(s * RPT, RPT)])


SUP = 8
NCH = EP // CHUNK
CPT = NCH // NS


@functools.partial(
    pl.kernel,
    mesh=_MESH,
    out_type=jax.ShapeDtypeStruct((NZ, D), jnp.float32),
    scratch_types=[
        pltpu.VMEM((SUP, CHUNK), jnp.int32),
        pltpu.VMEM((SUP, CHUNK), jnp.int32),
        pltpu.VMEM((CHUNK, D), jnp.float32),
        pltpu.VMEM((CHUNK, D), jnp.float32),
        pltpu.VMEM_SHARED((NZ, D), jnp.float32),
        pltpu.SemaphoreType.DMA,
        pltpu.SemaphoreType.DMA,
    ],
)
def _edge_kernel(src_hbm, dst_hbm, y_hbm, out_hbm,
                 src_v, dst_v, rows_a, rows_b, z_sh, sem_a, sem_b):
    c = lax.axis_index("c")
    s = lax.axis_index("s")

    def gather(j, buf, sem):
        return pltpu.make_async_copy(y_hbm.at[src_v.at[j]], buf, sem)

    def group(g, carry):
        base = s * CPT + g * SUP
        pltpu.sync_copy(src_hbm.at[pl.ds(base, SUP)], src_v)
        pltpu.sync_copy(dst_hbm.at[pl.ds(base, SUP)], dst_v)
        gather(0, rows_a, sem_a).start()
        gather(1, rows_b, sem_b).start()

        def body(i, carry2):
            j0 = 2 * i
            j1 = 2 * i + 1
            gather(j0, rows_a, sem_a).wait()
            pltpu.sync_copy(rows_a, z_sh.at[dst_v.at[j0]], add=True)

            @pl.when(j0 + 2 < SUP)
            def _():
                gather(j0 + 2, rows_a, sem_a).start()

            gather(j1, rows_b, sem_b).wait()
            pltpu.sync_copy(rows_b, z_sh.at[dst_v.at[j1]], add=True)

            @pl.when(j1 + 2 < SUP)
            def _():
                gather(j1 + 2, rows_b, sem_b).start()

            return carry2

        lax.fori_loop(0, SUP // 2, body, None)
        return carry

    @pl.when(c == 0)
    def _():
        def zrow(i, carry):
            for l in range(D // 16):
                rows_a[i, pl.ds(l * 16, 16)] = jnp.zeros((16,), jnp.float32)
            return carry

        lax.fori_loop(0, CHUNK, zrow, None)

        def zero_sh(i, carry):
            pltpu.sync_copy(rows_a,
                            z_sh.at[pl.ds(s * RPT + i * CHUNK, CHUNK)])
            return carry

        lax.fori_loop(0, RPT // CHUNK, zero_sh, None)
        plsc.subcore_barrier()
        lax.fori_loop(0, CPT // SUP, group, None)
        plsc.subcore_barrier()
        pltpu.sync_copy(z_sh.at[pl.ds(s * RPT, RPT)],
                        out_hbm.at[pl.ds(s * RPT, RPT)])


BM = 320
GB = NZ // BM
BMO = 400
GBO = N // BMO


def _mm1_body(degp_ref, x_ref, w_ref, y_ref, dinv_ref):
    degp = degp_ref[...]
    deg = degp[0, :, 0] + degp[1, :, 0] + 1.0
    dinv = lax.rsqrt(deg)
    xw = jnp.dot(x_ref[...], w_ref[...], preferred_element_type=jnp.float32)
    y_ref[...] = xw * dinv[:, None]
    dinv_ref[...] = dinv[:, None]


def _mm2_body(zp_ref, y1_ref, dinv_ref, b1_ref, w2_ref, y2_ref):
    dinv = dinv_ref[...]
    t = (zp_ref[...] + y1_ref[...]) * dinv + b1_ref[...]
    h = jnp.maximum(t, 0.0)
    y2_ref[...] = jnp.dot(h, w2_ref[...],
                          preferred_element_type=jnp.float32) * dinv


def _out_body(zp_ref, y2_ref, dinv_ref, b2_ref, o_ref):
    o_ref[...] = (zp_ref[...] + y2_ref[...]) * dinv_ref[...] + b2_ref[...]


_mm1 = pl.pallas_call(
    _mm1_body,
    grid=(GB,),
    in_specs=[
        pl.BlockSpec((NC, BM, 16), lambda i: (0, i, 0)),
        pl.BlockSpec((BM, D), lambda i: (i, 0)),
        pl.BlockSpec((D, D), lambda i: (0, 0)),
    ],
    out_specs=[
        pl.BlockSpec((BM, D), lambda i: (i, 0)),
        pl.BlockSpec((BM, 1), lambda i: (i, 0)),
    ],
    out_shape=[
        jax.ShapeDtypeStruct((NZ, D), jnp.float32),
        jax.ShapeDtypeStruct((NZ, 1), jnp.float32),
    ],
)

_mm2 = pl.pallas_call(
    _mm2_body,
    grid=(GB,),
    in_specs=[
        pl.BlockSpec((BM, D), lambda i: (i, 0)),
        pl.BlockSpec((BM, D), lambda i: (i, 0)),
        pl.BlockSpec((BM, 1), lambda i: (i, 0)),
        pl.BlockSpec((1, D), lambda i: (0, 0)),
        pl.BlockSpec((D, D), lambda i: (0, 0)),
    ],
    out_specs=pl.BlockSpec((BM, D), lambda i: (i, 0)),
    out_shape=jax.ShapeDtypeStruct((NZ, D), jnp.float32),
)

_out = pl.pallas_call(
    _out_body,
    grid=(GBO,),
    in_specs=[
        pl.BlockSpec((BMO, D), lambda i: (i, 0)),
        pl.BlockSpec((BMO, D), lambda i: (i, 0)),
        pl.BlockSpec((BMO, 1), lambda i: (i, 0)),
        pl.BlockSpec((1, D), lambda i: (0, 0)),
    ],
    out_specs=pl.BlockSpec((BMO, D), lambda i: (i, 0)),
    out_shape=jax.ShapeDtypeStruct((N, D), jnp.float32),
)


def kernel(x, edge_index, W1, b1, W2, b2):
    src = edge_index[0]
    dst = edge_index[1]
    pad = EP - E
    x_p = jnp.concatenate([x, jnp.zeros((NZ - N, D), jnp.float32)])
    src_fill = (jnp.arange(pad, dtype=jnp.int32) * 131) % N
    src_p = jnp.concatenate([src, src_fill]).reshape(EP // CHUNK, CHUNK)
    dump = N + jnp.arange(pad, dtype=jnp.int32) % (NZ - N)
    dst_p = jnp.concatenate([dst, dump]).reshape(EP // CHUNK, CHUNK)

    degp = _deg_kernel(dst_p)
    y1, dinv = _mm1(degp, x_p, W1)
    z1 = _edge_kernel(src_p, dst_p, y1)
    y2 = _mm2(z1, y1, dinv, b1.reshape(1, D), W2)
    z2 = _edge_kernel(src_p, dst_p, y2)
    return _out(z2, y2, dinv, b2.reshape(1, D))
